# Initial kernel scaffold; baseline (speedup 1.0000x reference)
import jax
import jax.numpy as jnp
from jax.experimental import pallas as pl


def kernel(harddecision, r):
    raise NotImplementedError("write your pallas kernel here")


# speedup vs baseline: 2.6787x; 2.6787x over previous
import functools

import numpy as np
import jax
import jax.numpy as jnp
from jax import lax
from jax.experimental import pallas as pl
from jax.experimental.pallas import tpu as pltpu
from jax.experimental.pallas import tpu_sc as plsc

_CODE = np.array(
    [[0, 0, 0, 0, 0, 0, 0], [1, 1, 1, 0, 0, 0, 0], [1, 0, 0, 1, 1, 0, 0],
     [0, 1, 1, 1, 1, 0, 0], [0, 1, 0, 1, 0, 1, 0], [1, 0, 1, 1, 0, 1, 0],
     [1, 1, 0, 0, 1, 1, 0], [0, 0, 1, 0, 1, 1, 0], [1, 1, 0, 1, 0, 0, 1],
     [0, 0, 1, 1, 0, 0, 1], [0, 1, 0, 0, 1, 0, 1], [1, 0, 1, 0, 1, 0, 1],
     [1, 0, 0, 0, 0, 1, 1], [0, 1, 1, 0, 0, 1, 1], [0, 0, 0, 1, 1, 1, 1],
     [1, 1, 1, 1, 1, 1, 1]], dtype=np.int32)
_P = np.arange(128)
_WORDS = ((_P[:, None] >> np.arange(7)[None, :]) & 1).astype(np.int32)
_NEAREST = _CODE[((_WORDS[:, None, :] != _CODE[None, :, :]).sum(2)).argmin(1)]
_LUT_FLAT = np.ascontiguousarray(_NEAREST.reshape(-1))

_NC, _NS, _L = 2, 16, 16
_NW = _NC * _NS


def _decode_kernel(B):
    n = B // _NW
    nw7, nw4 = n * 7, n * 4
    mesh = plsc.VectorSubcoreMesh(core_axis_name="c", subcore_axis_name="s")

    @functools.partial(
        pl.kernel,
        mesh=mesh,
        out_type=jax.ShapeDtypeStruct((B * 4,), jnp.int32),
        compiler_params=pltpu.CompilerParams(needs_layout_passes=False),
        scratch_types=[
            pltpu.VMEM((128 * 7,), jnp.int32),
            pltpu.VMEM((32,), jnp.float32),
            pltpu.VMEM((512,), jnp.int32),
            pltpu.VMEM((nw7,), jnp.int32),
            pltpu.VMEM((nw4,), jnp.int32),
        ],
    )
    def k(lut_hbm, r_hbm, hd_hbm, out_hbm, lut_v, r_v, tbl_v, hd_v, out_v):
        wid = lax.axis_index("s") * _NC + lax.axis_index("c")
        lane = lax.iota(jnp.int32, _L)
        lane7 = lane * 7
        lane4 = lane * 4

        pltpu.sync_copy(lut_hbm, lut_v)
        pltpu.sync_copy(r_hbm, r_v)

        for s in range(512 // _L):
            m = lane + s * _L
            c7 = (m >> 2) * 7
            j7 = (m & 3) * 7
            acc = jnp.zeros((_L,), jnp.float32)
            for kk in range(7):
                bk = plsc.load_gather(lut_v, [c7 + kk])
                rv = plsc.load_gather(r_v, [j7 + kk])
                acc = acc + bk.astype(jnp.float32) * rv
            tbl_v[pl.ds(s * _L, _L)] = acc.astype(jnp.int32)

        pltpu.sync_copy(hd_hbm.at[pl.ds(wid * nw7, nw7)], hd_v)

        def step(i, carry):
            base = i * (7 * _L)
            idx = plsc.load_gather(hd_v, [lane7 + base])
            for kk in range(1, 7):
                bk = plsc.load_gather(hd_v, [lane7 + (base + kk)])
                idx = idx + (bk << kk)
            t4 = idx * 4
            ob = i * (4 * _L)
            for j in range(4):
                vj = plsc.load_gather(tbl_v, [t4 + j])
                plsc.store_scatter(out_v, [lane4 + (ob + j)], vj)
            return carry

        lax.fori_loop(0, n // _L, step, 0)
        pltpu.sync_copy(out_v, out_hbm.at[pl.ds(wid * nw4, nw4)])

    return k


def kernel(harddecision, r):
    B = harddecision.shape[0]
    hd_flat = harddecision.reshape(-1)
    r_flat = jnp.pad(r.reshape(-1), (0, 4))
    out = _decode_kernel(B)(jnp.asarray(_LUT_FLAT), r_flat, hd_flat)
    return out.reshape(B, 4)
